# baseline (speedup 1.0000x reference)
import jax
import jax.numpy as jnp
from jax import lax
from jax.experimental import pallas as pl
from jax.experimental.pallas import tpu as pltpu
from jax.experimental.pallas import tpu_sc as plsc

B = 16384
F = 26
ROWS_PER_FIELD = 40000
NC = 2
NS = 16
BH = B // NC
SEG = BH // NS
LANES = 16


def _body(x_hbm, table_hbm, bias_hbm, out_hbm,
          x_v, idx0_v, idx1_v, emb_v, acc_v, seg_v, tmp_v, bias_v, part_s, sem):
    c = lax.axis_index("c")
    s = lax.axis_index("s")
    bbase = c * BH
    two = s < F - NS

    pltpu.sync_copy(bias_hbm, bias_v)

    pltpu.sync_copy(x_hbm.at[s, pl.ds(bbase, BH)], x_v.at[0])

    @pl.when(two)
    def _stage2():
        pltpu.sync_copy(x_hbm.at[s + NS, pl.ds(bbase, BH)], x_v.at[1])

    def build(k, idx_ref, off):
        def grp(i, _):
            col = i * LANES
            idx_ref[pl.ds(col, LANES)] = x_v[k, pl.ds(col, LANES)] + off
            return 0

        lax.fori_loop(0, BH // LANES, grp, 0)

    build(0, idx0_v, s * ROWS_PER_FIELD)
    pltpu.async_copy(table_hbm.at[idx0_v], acc_v, sem)

    @pl.when(two)
    def _fire2():
        build(1, idx1_v, (s + NS) * ROWS_PER_FIELD)
        pltpu.async_copy(table_hbm.at[idx1_v], emb_v, sem)

    pltpu.make_async_copy(table_hbm.at[idx0_v], acc_v, sem).wait()

    @pl.when(two)
    def _drain2():
        pltpu.make_async_copy(table_hbm.at[idx1_v], emb_v, sem).wait()

        def addemb(i, _):
            col = i * LANES
            acc_v[pl.ds(col, LANES)] = acc_v[pl.ds(col, LANES)] + emb_v[pl.ds(col, LANES)]
            return 0

        lax.fori_loop(0, BH // LANES, addemb, 0)

    pltpu.sync_copy(acc_v, part_s.at[s])
    plsc.subcore_barrier()

    segbase = s * SEG
    pltpu.sync_copy(part_s.at[0, pl.ds(segbase, SEG)], seg_v)

    def addpart(t, _):
        pltpu.sync_copy(part_s.at[t, pl.ds(segbase, SEG)], tmp_v)
        for g in range(SEG // LANES):
            col = g * LANES
            seg_v[pl.ds(col, LANES)] = seg_v[pl.ds(col, LANES)] + tmp_v[pl.ds(col, LANES)]
        return 0

    lax.fori_loop(1, NS, addpart, 0)

    bias_vec = bias_v[...]
    for g in range(SEG // LANES):
        col = g * LANES
        seg_v[pl.ds(col, LANES)] = seg_v[pl.ds(col, LANES)] + bias_vec

    pltpu.sync_copy(seg_v, out_hbm.at[pl.ds(bbase + segbase, SEG)])


def kernel(x, table, bias):
    xf = x.astype(jnp.int32).T
    tf = table.reshape(-1)
    bb = jnp.tile(bias.astype(jnp.float32), LANES)
    run = pl.kernel(
        _body,
        mesh=plsc.VectorSubcoreMesh(core_axis_name="c", subcore_axis_name="s"),
        out_type=jax.ShapeDtypeStruct((B,), jnp.float32),
        scratch_types=[
            pltpu.VMEM((2, BH), jnp.int32),
            pltpu.VMEM((BH,), jnp.int32),
            pltpu.VMEM((BH,), jnp.int32),
            pltpu.VMEM((BH,), jnp.float32),
            pltpu.VMEM((BH,), jnp.float32),
            pltpu.VMEM((SEG,), jnp.float32),
            pltpu.VMEM((SEG,), jnp.float32),
            pltpu.VMEM((LANES,), jnp.float32),
            pltpu.VMEM_SHARED((NS, BH), jnp.float32),
            pltpu.SemaphoreType.DMA,
        ],
    )
    out = run(xf, tf, bb)
    return out.reshape(B, 1)


# speedup vs baseline: 1.1321x; 1.1321x over previous
import jax
import jax.numpy as jnp
from jax import lax
from jax.experimental import pallas as pl
from jax.experimental.pallas import tpu as pltpu
from jax.experimental.pallas import tpu_sc as plsc

B = 16384
F = 26
ROWS_PER_FIELD = 40000
NC = 2
NS = 16
NW = NC * NS
BPW = B // NW
LANES = 16
GPF = BPW // LANES


def _body(x_hbm, table_hbm, bias_hbm, out_hbm, x_v, idx_v, emb_v, out_v, bias_v, sem):
    wid = lax.axis_index("s") * NC + lax.axis_index("c")
    base = wid * BPW

    pltpu.sync_copy(x_hbm.at[:, pl.ds(base, BPW)], x_v)
    pltpu.sync_copy(bias_hbm, bias_v)

    def step(f, _):
        off = f * ROWS_PER_FIELD
        for g in range(GPF):
            idx_v[pl.ds(f * BPW + g * LANES, LANES)] = (
                x_v[f, pl.ds(g * LANES, LANES)] + off
            )
        pltpu.async_copy(
            table_hbm.at[idx_v.at[pl.ds(f * BPW, BPW)]],
            emb_v.at[pl.ds(f * BPW, BPW)],
            sem,
        )
        return 0

    lax.fori_loop(0, F, step, 0)
    pltpu.make_async_copy(table_hbm.at[pl.ds(0, F * BPW)], emb_v, sem).wait()

    bias_vec = bias_v[...]

    def red(s, _):
        col = s * LANES
        acc = bias_vec
        for f in range(F):
            acc = acc + emb_v[pl.ds(f * BPW + col, LANES)]
        out_v[pl.ds(col, LANES)] = acc
        return 0

    lax.fori_loop(0, GPF, red, 0)

    pltpu.sync_copy(out_v, out_hbm.at[pl.ds(base, BPW)])


def kernel(x, table, bias):
    xf = x.astype(jnp.int32).T
    tf = table.reshape(-1)
    bb = jnp.tile(bias.astype(jnp.float32), LANES)
    run = pl.kernel(
        _body,
        mesh=plsc.VectorSubcoreMesh(core_axis_name="c", subcore_axis_name="s"),
        out_type=jax.ShapeDtypeStruct((B,), jnp.float32),
        scratch_types=[
            pltpu.VMEM((F, BPW), jnp.int32),
            pltpu.VMEM((F * BPW,), jnp.int32),
            pltpu.VMEM((F * BPW,), jnp.float32),
            pltpu.VMEM((BPW,), jnp.float32),
            pltpu.VMEM((LANES,), jnp.float32),
            pltpu.SemaphoreType.DMA,
        ],
    )
    out = run(xf, tf, bb)
    return out.reshape(B, 1)


# speedup vs baseline: 2.2639x; 1.9997x over previous
import jax
import jax.numpy as jnp
from jax import lax
from jax.experimental import pallas as pl
from jax.experimental.pallas import tpu as pltpu
from jax.experimental.pallas import tpu_sc as plsc

B = 16384
F = 26
ROWS_PER_FIELD = 40000
NC = 2
NS = 16
NW = NC * NS
BPW = B // NW
LANES = 16
GPF = BPW // LANES


def _body(x_hbm, table_hbm, bias_hbm, out_hbm, x_v, idx_v, emb_v, out_v, bias_v, sem):
    wid = lax.axis_index("s") * NC + lax.axis_index("c")
    base = wid * BPW

    pltpu.sync_copy(x_hbm.at[:, pl.ds(base, BPW)], x_v)
    pltpu.sync_copy(bias_hbm, bias_v)

    def step(f, _):
        off = f * ROWS_PER_FIELD
        for g in range(GPF):
            idx_v[pl.ds(f * BPW + g * LANES, LANES)] = (
                x_v[f, pl.ds(g * LANES, LANES)] + off
            )
        pltpu.async_copy(
            table_hbm.at[0].at[idx_v.at[pl.ds(f * BPW, BPW)]],
            emb_v.at[pl.ds(f * BPW, BPW)],
            sem,
        )
        return 0

    lax.fori_loop(0, F, step, 0)
    pltpu.make_async_copy(table_hbm.at[0].at[pl.ds(0, F * BPW)], emb_v, sem).wait()

    bias_vec = bias_v[...]

    def red(s, _):
        col = s * LANES
        acc = bias_vec
        for f in range(F):
            acc = acc + emb_v[pl.ds(f * BPW + col, LANES)]
        out_v[pl.ds(col, LANES)] = acc
        return 0

    lax.fori_loop(0, GPF, red, 0)

    pltpu.sync_copy(out_v, out_hbm.at[pl.ds(base, BPW)])


def kernel(x, table, bias):
    xf = x.astype(jnp.int32).T
    bb = jnp.tile(bias.astype(jnp.float32), LANES)
    run = pl.kernel(
        _body,
        mesh=plsc.VectorSubcoreMesh(core_axis_name="c", subcore_axis_name="s"),
        out_type=jax.ShapeDtypeStruct((B,), jnp.float32),
        scratch_types=[
            pltpu.VMEM((F, BPW), jnp.int32),
            pltpu.VMEM((F * BPW,), jnp.int32),
            pltpu.VMEM((F * BPW,), jnp.float32),
            pltpu.VMEM((BPW,), jnp.float32),
            pltpu.VMEM((LANES,), jnp.float32),
            pltpu.SemaphoreType.DMA,
        ],
    )
    out = run(xf, table.T, bb)
    return out.reshape(B, 1)
